# baseline (speedup 1.0000x reference)
import jax
import jax.numpy as jnp
from jax import lax
from jax.experimental import pallas as pl
from jax.experimental.pallas import tpu as pltpu
from jax.experimental.pallas import tpu_sc as plsc

_BATCH = 16384
_N_FIELDS = 26
_VOCAB = 100000
_EMB_DIM = 16
_N_CONT = 10
_OUT_W = _N_FIELDS * _EMB_DIM + _N_CONT

_NC = 2
_NS = 16
_LANES = 16
_NW = _NC * _NS
_BPW = _BATCH // _NW
_GRP = _BPW // _LANES
_CSL = _BPW * _N_CONT // _LANES


def _body(xcatt_hbm, xcont_hbm, cmin_hbm, cmax_hbm, *rest):
    tabs = rest[:_N_FIELDS]
    out_hbm = rest[_N_FIELDS]
    (xc_v, idx0_v, idx1_v, fc0_v, fc1_v,
     cin_v, cout_v, cm_v, cx_v, sem0, sem1) = rest[_N_FIELDS + 1:]

    wid = lax.axis_index("s") * _NC + lax.axis_index("c")
    base = wid * _BPW
    iota = lax.iota(jnp.int32, _LANES)

    pltpu.sync_copy(xcatt_hbm.at[:, pl.ds(base, _BPW)], xc_v)

    idxv = (idx0_v, idx1_v)
    fcv = (fc0_v, fc1_v)
    sems = (sem0, sem1)
    desc = [None, None]

    def build_idx(f):
        idxr = idxv[f & 1]
        fvec = jnp.full((_LANES,), f, jnp.int32)

        @pl.loop(0, _GRP)
        def _(g):
            rvec = g * _LANES + iota
            idxr[pl.ds(g * _LANES, _LANES)] = plsc.load_gather(
                xc_v, [fvec, rvec])

    def cont_path():
        pltpu.sync_copy(xcont_hbm.at[pl.ds(base, _BPW)], cin_v)
        pltpu.sync_copy(cmin_hbm, cm_v)
        pltpu.sync_copy(cmax_hbm, cx_v)

        @pl.loop(0, _CSL)
        def _(j):
            p = j * _LANES + iota
            r = p // _N_CONT
            c = p % _N_CONT
            x = plsc.load_gather(cin_v, [r, c])
            mn = plsc.load_gather(cm_v, [c])
            mx = plsc.load_gather(cx_v, [c])
            plsc.store_scatter(cout_v, [r, c], (x - mn) / (mx - mn))

        pltpu.sync_copy(
            cout_v,
            out_hbm.at[pl.ds(base, _BPW),
                       pl.ds(_N_FIELDS * _EMB_DIM, _N_CONT)])

    for f in range(_N_FIELDS):
        cur = f & 1
        build_idx(f)
        desc[cur] = pltpu.async_copy(tabs[f].at[idxv[cur]], fcv[cur],
                                     sems[cur])
        if f == 0:
            cont_path()
        if f >= 1:
            prev = 1 - cur
            desc[prev].wait()
            pltpu.sync_copy(
                fcv[prev],
                out_hbm.at[pl.ds(base, _BPW),
                           pl.ds((f - 1) * _EMB_DIM, _EMB_DIM)])
    desc[1].wait()
    pltpu.sync_copy(
        fcv[1],
        out_hbm.at[pl.ds(base, _BPW),
                   pl.ds((_N_FIELDS - 1) * _EMB_DIM, _EMB_DIM)])


_mesh = plsc.VectorSubcoreMesh(core_axis_name="c", subcore_axis_name="s")

_sc_call = pl.kernel(
    _body,
    out_type=jax.ShapeDtypeStruct((_BATCH, _OUT_W), jnp.float32),
    mesh=_mesh,
    scratch_types=[
        pltpu.VMEM((_N_FIELDS, _BPW), jnp.int32),
        pltpu.VMEM((_BPW,), jnp.int32),
        pltpu.VMEM((_BPW,), jnp.int32),
        pltpu.VMEM((_BPW, _EMB_DIM), jnp.float32),
        pltpu.VMEM((_BPW, _EMB_DIM), jnp.float32),
        pltpu.VMEM((_BPW, _N_CONT), jnp.float32),
        pltpu.VMEM((_BPW, _N_CONT), jnp.float32),
        pltpu.VMEM((_N_CONT,), jnp.float32),
        pltpu.VMEM((_N_CONT,), jnp.float32),
        pltpu.SemaphoreType.DMA,
        pltpu.SemaphoreType.DMA,
    ],
    compiler_params=pltpu.CompilerParams(
        use_tc_tiling_on_sc=False, needs_layout_passes=False),
)


@jax.jit
def kernel(x_cat, x_cont, tables, cont_min, cont_max):
    xcatt = x_cat.astype(jnp.int32).T
    tabs = [tables[f] for f in range(_N_FIELDS)]
    return _sc_call(xcatt, x_cont, cont_min, cont_max, *tabs)


# speedup vs baseline: 1.5441x; 1.5441x over previous
import jax
import jax.numpy as jnp
from jax import lax
from jax.experimental import pallas as pl
from jax.experimental.pallas import tpu as pltpu
from jax.experimental.pallas import tpu_sc as plsc

_BATCH = 16384
_N_FIELDS = 26
_VOCAB = 100000
_EMB_DIM = 16
_N_CONT = 10
_OUT_W = _N_FIELDS * _EMB_DIM + _N_CONT

_NC = 2
_NS = 16
_LANES = 16
_NW = _NC * _NS
_BPW = _BATCH // _NW
_GRP = _BPW // _LANES
_CSL = _BPW * _N_CONT // _LANES


def _body(tab_hbm, xcatt_hbm, xcont_hbm, cmin_hbm, cmax_hbm, out_hbm,
          xc_v, idx0_v, idx1_v, fc0_v, fc1_v,
          cin_v, cout_v, cm_v, cx_v, sem0, sem1):
    wid = lax.axis_index("s") * _NC + lax.axis_index("c")
    base = wid * _BPW
    iota = lax.iota(jnp.int32, _LANES)

    pltpu.sync_copy(xcatt_hbm.at[:, pl.ds(base, _BPW)], xc_v)

    idxv = (idx0_v, idx1_v)
    fcv = (fc0_v, fc1_v)
    sems = (sem0, sem1)
    desc = [None, None]

    def build_idx(f):
        idxr = idxv[f & 1]
        fvec = jnp.full((_LANES,), f, jnp.int32)

        @pl.loop(0, _GRP)
        def _(g):
            rvec = g * _LANES + iota
            col = plsc.load_gather(xc_v, [fvec, rvec])
            idxr[pl.ds(g * _LANES, _LANES)] = col + f * _VOCAB

    def cont_path():
        pltpu.sync_copy(xcont_hbm.at[pl.ds(base, _BPW)], cin_v)
        pltpu.sync_copy(cmin_hbm, cm_v)
        pltpu.sync_copy(cmax_hbm, cx_v)

        @pl.loop(0, _CSL)
        def _(j):
            p = j * _LANES + iota
            r = p // _N_CONT
            c = p % _N_CONT
            x = plsc.load_gather(cin_v, [r, c])
            mn = plsc.load_gather(cm_v, [c])
            mx = plsc.load_gather(cx_v, [c])
            plsc.store_scatter(cout_v, [r, c], (x - mn) / (mx - mn))

        pltpu.sync_copy(
            cout_v,
            out_hbm.at[pl.ds(base, _BPW),
                       pl.ds(_N_FIELDS * _EMB_DIM, _N_CONT)])

    for f in range(_N_FIELDS):
        cur = f & 1
        build_idx(f)
        desc[cur] = pltpu.async_copy(tab_hbm.at[idxv[cur]], fcv[cur],
                                     sems[cur])
        if f == 0:
            cont_path()
        if f >= 1:
            prev = 1 - cur
            desc[prev].wait()
            pltpu.sync_copy(
                fcv[prev],
                out_hbm.at[pl.ds(base, _BPW),
                           pl.ds((f - 1) * _EMB_DIM, _EMB_DIM)])
    desc[1].wait()
    pltpu.sync_copy(
        fcv[1],
        out_hbm.at[pl.ds(base, _BPW),
                   pl.ds((_N_FIELDS - 1) * _EMB_DIM, _EMB_DIM)])


_mesh = plsc.VectorSubcoreMesh(core_axis_name="c", subcore_axis_name="s")

_sc_call = pl.kernel(
    _body,
    out_type=jax.ShapeDtypeStruct((_BATCH, _OUT_W), jnp.float32),
    mesh=_mesh,
    scratch_types=[
        pltpu.VMEM((_N_FIELDS, _BPW), jnp.int32),
        pltpu.VMEM((_BPW,), jnp.int32),
        pltpu.VMEM((_BPW,), jnp.int32),
        pltpu.VMEM((_BPW, _EMB_DIM), jnp.float32),
        pltpu.VMEM((_BPW, _EMB_DIM), jnp.float32),
        pltpu.VMEM((_BPW, _N_CONT), jnp.float32),
        pltpu.VMEM((_BPW, _N_CONT), jnp.float32),
        pltpu.VMEM((_N_CONT,), jnp.float32),
        pltpu.VMEM((_N_CONT,), jnp.float32),
        pltpu.SemaphoreType.DMA,
        pltpu.SemaphoreType.DMA,
    ],
    compiler_params=pltpu.CompilerParams(
        use_tc_tiling_on_sc=False, needs_layout_passes=False),
)


@jax.jit
def kernel(x_cat, x_cont, tables, cont_min, cont_max):
    xcatt = x_cat.astype(jnp.int32).T
    tab_flat = tables.reshape(_N_FIELDS * _VOCAB, _EMB_DIM)
    return _sc_call(tab_flat, xcatt, x_cont, cont_min, cont_max)
